# Initial kernel scaffold; baseline (speedup 1.0000x reference)
import jax
import jax.numpy as jnp
from jax.experimental import pallas as pl


def kernel(x, edge_index, batch, index_matrix_0, weight_matrix_0, pooled_edge_index_0, batch_1, gat0_Wl, gat0_Wr, gat0_att, gat0_b, gat1_Wl, gat1_Wr, gat1_att, gat1_b, gat2_Wl, gat2_Wr, gat2_att, gat2_b, gat3_Wl, gat3_Wr, gat3_att, gat3_b, gcn_Wl, gcn_Wr, gcn_att, gcn_b, gate_W, gate_b, cls_W, cls_b):
    raise NotImplementedError("write your pallas kernel here")


# speedup vs baseline: 9.3035x; 9.3035x over previous
import jax
import jax.numpy as jnp
from jax import lax
from jax.experimental import pallas as pl
from jax.experimental.pallas import tpu as pltpu
from jax.experimental.pallas import tpu_sc as plsc

_N = 10000
_D = 128
_E_TOT = 330000
_N_PAD = 10240
_NP = 5000
_NP_PAD = 5120
_DP = 64
_EP_TOT = 165000
_K = 8
_G = 16
_CHUNK = 80
_NC = 2
_NS = 16
_NW = _NC * _NS
_ZRN = 80


def _cdiv(a, b):
    return (a + b - 1) // b


def _mesh():
    return plsc.VectorSubcoreMesh(
        core_axis_name="c", subcore_axis_name="s", num_cores=_NC,
        num_subcores=_NS,
    )


def _make_edge_kernel(n_pad, d, e_tot):
    cpw = _cdiv(e_tot, _NW * _CHUNK)
    e_pad = cpw * _NW * _CHUNK
    nd = n_pad // 8
    npd = n_pad + nd
    rt = n_pad // _NS
    rtd = nd // _NS
    zr = min(rt, _ZRN)
    while rt % zr or rtd % min(zr, rtd):
        zr -= 1
    nzc = rt // zr
    nk = d // 16

    def body(xl_hbm, xr_hbm, src_hbm, dst_hbm, att_hbm, zn_hbm,
             numer_out,
             att_v, src_v, dst_v, dd_v, xlr, xrr, tbuf, accb, dbuf,
             numer_sh, gsem, gsem2):
        c = lax.axis_index("c")
        s = lax.axis_index("s")
        w = c * _NS + s

        for j in range(nzc):
            pltpu.sync_copy(zn_hbm.at[pl.ds(0, zr)],
                            numer_sh.at[pl.ds(s * rt + j * zr, zr)])
        for j in range(_cdiv(rtd, zr)):
            rows = min(zr, rtd - j * zr)
            pltpu.sync_copy(zn_hbm.at[pl.ds(0, rows)],
                            numer_sh.at[pl.ds(n_pad + s * rtd + j * zr, rows)])
        pltpu.sync_copy(att_hbm, att_v)
        plsc.subcore_barrier()

        def chunk_body(j, carry):
            base = (w * cpw + j) * _CHUNK
            pltpu.sync_copy(src_hbm.at[pl.ds(base, _CHUNK)], src_v)
            pltpu.sync_copy(dst_hbm.at[pl.ds(base, _CHUNK)], dst_v)
            cp1 = pltpu.async_copy(xl_hbm.at[src_v], xlr, gsem)
            cp2 = pltpu.async_copy(xr_hbm.at[dst_v], xrr, gsem2)
            for q in range(_CHUNK // 16):
                dv = dst_v[pl.ds(q * 16, 16)]
                dd_v[pl.ds(q * 16, 16)] = n_pad + lax.shift_right_logical(dv, 3)
            cp1.wait()
            cp2.wait()

            def edge_body(e, carry2):
                acc = jnp.zeros((16,), jnp.float32)
                for k in range(nk):
                    a = xlr[e, pl.ds(k * 16, 16)]
                    r = xrr[e, pl.ds(k * 16, 16)]
                    v = a + r
                    v = jnp.maximum(v, 0.2 * v)
                    acc = acc + v * att_v[pl.ds(k * 16, 16)]
                accb[pl.ds(e * 16, 16)] = acc
                return carry2
            lax.fori_loop(0, _CHUNK, edge_body, 0)

            iota = lax.iota(jnp.int32, 16)

            def group_body(g, carry2):
                rows = (g * 16 + iota) * 16
                tot = jnp.zeros((16,), jnp.float32)
                for col in range(16):
                    tot = tot + plsc.load_gather(accb, [rows + col])
                valid = (base + g * 16 + iota) < e_tot
                t16 = jnp.where(valid, jnp.exp(tot), jnp.float32(0.0))
                dstg = dst_v[pl.ds(g * 16, 16)]
                for el in range(16):
                    t_sc = t16[el]
                    e = g * 16 + el
                    for k in range(nk):
                        xlr[e, pl.ds(k * 16, 16)] = (
                            xlr[e, pl.ds(k * 16, 16)] * t_sc)
                    slot = dstg[el] & 7
                    tb16 = jnp.broadcast_to(t_sc, (16,))
                    for sl in range(8):
                        on = jnp.where(sl == slot, jnp.float32(1.0),
                                       jnp.float32(0.0))
                        tbuf[e, pl.ds(sl * 16, 16)] = tb16 * on
                return carry2
            lax.fori_loop(0, _CHUNK // 16, group_body, 0)
            pltpu.sync_copy(xlr, numer_sh.at[dst_v], add=True)
            pltpu.sync_copy(tbuf, numer_sh.at[dd_v], add=True)
            return carry
        lax.fori_loop(0, cpw, chunk_body, 0)
        plsc.subcore_barrier()
        pltpu.sync_copy(numer_sh.at[pl.ds(s * rt, rt)],
                        numer_out.at[pl.ds(c * 2 * n_pad + s * rt, rt)])
        for jj in range(rt // _CHUNK):
            pltpu.sync_copy(
                numer_sh.at[pl.ds(n_pad + s * rtd + jj * (_CHUNK // 8),
                                  _CHUNK // 8)], dbuf)

            def expand_body(n, carry):
                r = lax.shift_right_logical(n, 3)
                sl = n & 7
                dchunk = dbuf[r, pl.ds(sl * 16, 16)]
                b16 = jnp.broadcast_to(dchunk[0], (16,))
                for k in range(nk):
                    tbuf[n, pl.ds(k * 16, 16)] = b16
                return carry
            lax.fori_loop(0, _CHUNK, expand_body, 0)
            pltpu.sync_copy(
                tbuf,
                numer_out.at[pl.ds(
                    c * 2 * n_pad + n_pad + s * rt + jj * _CHUNK, _CHUNK)])

    kfn = pl.kernel(
        body,
        out_type=jax.ShapeDtypeStruct((_NC * 2 * n_pad, d), jnp.float32),
        mesh=_mesh(),
        compiler_params=pltpu.CompilerParams(needs_layout_passes=False),
        scratch_types=[
            pltpu.VMEM((d,), jnp.float32),
            pltpu.VMEM((_CHUNK,), jnp.int32),
            pltpu.VMEM((_CHUNK,), jnp.int32),
            pltpu.VMEM((_CHUNK,), jnp.int32),
            pltpu.VMEM((_CHUNK, d), jnp.float32),
            pltpu.VMEM((_CHUNK, d), jnp.float32),
            pltpu.VMEM((_CHUNK, d), jnp.float32),
            pltpu.VMEM((_CHUNK * 16,), jnp.float32),
            pltpu.VMEM((_CHUNK // 8, d), jnp.float32),
            pltpu.VMEM_SHARED((npd, d), jnp.float32),
            pltpu.SemaphoreType.DMA,
            pltpu.SemaphoreType.DMA,
        ],
    )
    return kfn, e_pad


def _make_pool_kernel():
    pw = _NP_PAD // _NW
    ncn = pw // 16
    nk = _D // 16

    def body(h_hbm, idx_hbm, w_hbm, out_hbm, idx_v, w_v, rows_v, hp_v):
        c = lax.axis_index("c")
        s = lax.axis_index("s")
        wkr = c * _NS + s
        nbase = wkr * pw
        pltpu.sync_copy(w_hbm.at[pl.ds(nbase * _K, pw * _K)], w_v)
        for cn in range(ncn):
            rbase = (nbase + cn * 16) * _K
            pltpu.sync_copy(idx_hbm.at[pl.ds(rbase, 16 * _K)], idx_v)
            pltpu.sync_copy(h_hbm.at[idx_v], rows_v)

            def pair_body(p, carry):
                wv = w_v[pl.ds(cn * 128 + p * 16, 16)]
                for half in range(2):
                    n = 2 * p + half
                    for dk in range(nk):
                        acc = jnp.zeros((16,), jnp.float32)
                        for k in range(_K):
                            acc = acc + (wv[half * _K + k]
                                         * rows_v[n * _K + k, pl.ds(dk * 16, 16)])
                        hp_v[n, pl.ds(dk * 16, 16)] = acc
                return carry
            lax.fori_loop(0, 8, pair_body, 0)
            pltpu.sync_copy(hp_v, out_hbm.at[pl.ds(nbase + cn * 16, 16)])

    return pl.kernel(
        body,
        out_type=jax.ShapeDtypeStruct((_NP_PAD, _D), jnp.float32),
        mesh=_mesh(),
        compiler_params=pltpu.CompilerParams(needs_layout_passes=False),
        scratch_types=[
            pltpu.VMEM((16 * _K,), jnp.int32),
            pltpu.VMEM((pw * _K,), jnp.float32),
            pltpu.VMEM((16 * _K, _D), jnp.float32),
            pltpu.VMEM((16, _D), jnp.float32),
        ],
    )


def _mm2_body(x_ref, wl_ref, wr_ref, xl_ref, xr_ref):
    x = x_ref[...]
    xl_ref[...] = jnp.dot(x, wl_ref[...], preferred_element_type=jnp.float32)
    xr_ref[...] = jnp.dot(x, wr_ref[...], preferred_element_type=jnp.float32)


def _mm2(x, wl, wr):
    n, d = x.shape
    dh = wl.shape[1]
    return pl.pallas_call(
        _mm2_body,
        out_shape=(jax.ShapeDtypeStruct((n, dh), jnp.float32),
                   jax.ShapeDtypeStruct((n, dh), jnp.float32)),
    )(x, wl, wr)


def _unpack(num, n, n_pad):
    outs = []
    for c in range(_NC):
        blk = num[c * 2 * n_pad:(c + 1) * 2 * n_pad]
        outs.append((blk[0:n], blk[n_pad:n_pad + n, 0:1]))
    return outs


def _make_combine_body(n, n_pad):
    def body(num_ref, b_ref, wl_ref, wr_ref, xl_ref, xr_ref):
        (n0, d0), (n1, d1) = _unpack(num_ref[...], n, n_pad)
        h = (n0 + n1) / (d0 + d1) + b_ref[...]
        h = jnp.maximum(h, 0.0)
        xl_ref[...] = jnp.dot(h, wl_ref[...], preferred_element_type=jnp.float32)
        xr_ref[...] = jnp.dot(h, wr_ref[...], preferred_element_type=jnp.float32)
    return body


def _combine(num, b, wl, wr, n, n_pad):
    dh = wl.shape[1]
    return pl.pallas_call(
        _make_combine_body(n, n_pad),
        out_shape=(jax.ShapeDtypeStruct((n, dh), jnp.float32),
                   jax.ShapeDtypeStruct((n, dh), jnp.float32)),
    )(num, b, wl, wr)


def _make_h4_body(n, n_pad):
    def body(num_ref, b_ref, h_ref):
        (n0, d0), (n1, d1) = _unpack(num_ref[...], n, n_pad)
        h = (n0 + n1) / (d0 + d1) + b_ref[...]
        h_ref[...] = jnp.where(h > 0, h, jnp.exp(h) - 1.0)
    return body


def _h4(num, b, n, n_pad, d):
    return pl.pallas_call(
        _make_h4_body(n, n_pad),
        out_shape=jax.ShapeDtypeStruct((n, d), jnp.float32),
    )(num, b)


def _final_body(num_ref, b_ref, batch_ref, gw_ref, gb_ref, cw_ref,
                cb_ref, out_ref):
    (n0, d0), (n1, d1) = _unpack(num_ref[...], _NP, _NP_PAD)
    h = (n0[:, 0:_DP] + n1[:, 0:_DP]) / (d0 + d1) + b_ref[...]
    h = jnp.where(h > 0, h, jnp.exp(h) - 1.0)
    gate = jnp.sum(h * gw_ref[...], axis=1) + gb_ref[0, 0]
    bvec = batch_ref[0, :]
    mask = bvec[None, :] == lax.broadcasted_iota(jnp.int32, (_G, _NP), 0)
    m = jnp.max(jnp.where(mask, gate[None, :], jnp.float32(-1e30)), axis=1)
    m = jnp.where(m < -1e29, 0.0, m)
    e = jnp.where(mask, jnp.exp(gate[None, :] - m[:, None]), 0.0)
    ssum = jnp.sum(e, axis=1)
    wn = e / (ssum[:, None] + 1e-16)
    pooled = jnp.dot(wn, h, preferred_element_type=jnp.float32)
    out_ref[...] = (jnp.dot(pooled, cw_ref[...], preferred_element_type=jnp.float32)
                    + cb_ref[...])


def _final(num, b, batch2, gw, gb, cw, cb):
    return pl.pallas_call(
        _final_body,
        out_shape=jax.ShapeDtypeStruct((_G, cw.shape[1]), jnp.float32),
    )(num, b, batch2, gw, gb, cw, cb)


def kernel(x, edge_index, batch, index_matrix_0, weight_matrix_0,
           pooled_edge_index_0, batch_1,
           gat0_Wl, gat0_Wr, gat0_att, gat0_b,
           gat1_Wl, gat1_Wr, gat1_att, gat1_b,
           gat2_Wl, gat2_Wr, gat2_att, gat2_b,
           gat3_Wl, gat3_Wr, gat3_att, gat3_b,
           gcn_Wl, gcn_Wr, gcn_att, gcn_b,
           gate_W, gate_b, cls_W, cls_b):
    i32 = jnp.int32
    edge_k, e_pad = _make_edge_kernel(_N_PAD, _D, _E_TOT)
    edge_kp, ep_pad = _make_edge_kernel(_NP_PAD, _D, _EP_TOT)
    pool_k = _make_pool_kernel()

    zn = jnp.zeros((_ZRN, _D), jnp.float32)
    si = jnp.arange(_N, dtype=i32)
    zpad = jnp.zeros((e_pad - _E_TOT,), i32)
    src = jnp.concatenate([edge_index[0].astype(i32), si, zpad])
    dst = jnp.concatenate([edge_index[1].astype(i32), si, zpad])

    gat_params = [(gat0_Wl, gat0_Wr, gat0_att, gat0_b),
                  (gat1_Wl, gat1_Wr, gat1_att, gat1_b),
                  (gat2_Wl, gat2_Wr, gat2_att, gat2_b),
                  (gat3_Wl, gat3_Wr, gat3_att, gat3_b)]

    xl, xr = _mm2(x, gat0_Wl, gat0_Wr)
    num = edge_k(xl, xr, src, dst, gat0_att, zn)
    for l in range(1, 4):
        wl, wr = gat_params[l][0], gat_params[l][1]
        b_prev = gat_params[l - 1][3].reshape(1, _D)
        xl, xr = _combine(num, b_prev, wl, wr, _N, _N_PAD)
        num = edge_k(xl, xr, src, dst, gat_params[l][2], zn)
    h = _h4(num, gat3_b.reshape(1, _D), _N, _N_PAD, _D)

    idx_flat = jnp.pad(index_matrix_0.astype(i32),
                       ((0, _NP_PAD - _NP), (0, 0))).reshape(-1)
    w_flat = jnp.pad(weight_matrix_0,
                     ((0, _NP_PAD - _NP), (0, 0))).reshape(-1)
    hp = pool_k(h, idx_flat, w_flat)[:_NP]

    gcn_wl_pad = jnp.pad(gcn_Wl, ((0, 0), (0, _D - _DP)))
    gcn_wr_pad = jnp.pad(gcn_Wr, ((0, 0), (0, _D - _DP)))
    xlp, xrp = _mm2(hp, gcn_wl_pad, gcn_wr_pad)
    sip = jnp.arange(_NP, dtype=i32)
    zpadp = jnp.zeros((ep_pad - _EP_TOT,), i32)
    srcp = jnp.concatenate([pooled_edge_index_0[0].astype(i32), sip, zpadp])
    dstp = jnp.concatenate([pooled_edge_index_0[1].astype(i32), sip, zpadp])
    nump = edge_kp(xlp, xrp, srcp, dstp,
                   jnp.pad(gcn_att, (0, _D - _DP)), zn)

    return _final(nump, gcn_b.reshape(1, _DP),
                  batch_1.astype(i32).reshape(1, _NP),
                  gate_W.reshape(1, _DP), gate_b.reshape(1, 1),
                  cls_W, cls_b.reshape(1, cls_W.shape[1]))


# speedup vs baseline: 10.0014x; 1.0750x over previous
import jax
import jax.numpy as jnp
from jax import lax
from jax.experimental import pallas as pl
from jax.experimental.pallas import tpu as pltpu
from jax.experimental.pallas import tpu_sc as plsc

_N = 10000
_D = 128
_E_TOT = 330000
_N_PAD = 10240
_NP = 5000
_NP_PAD = 5120
_DP = 64
_EP_TOT = 165000
_K = 8
_G = 16
_CHUNK = 80
_NC = 2
_NS = 16
_NW = _NC * _NS
_ZRN = 80


def _cdiv(a, b):
    return (a + b - 1) // b


def _mesh():
    return plsc.VectorSubcoreMesh(
        core_axis_name="c", subcore_axis_name="s", num_cores=_NC,
        num_subcores=_NS,
    )


def _make_edge_kernel(n_pad, d, e_tot):
    cpw = _cdiv(e_tot, _NW * _CHUNK)
    e_pad = cpw * _NW * _CHUNK
    nd = n_pad // 8
    npd = n_pad + nd
    rt = n_pad // _NS
    rtd = nd // _NS
    zr = min(rt, _ZRN)
    while rt % zr or rtd % min(zr, rtd):
        zr -= 1
    nzc = rt // zr
    nk = d // 16

    def body(xl_hbm, xr_hbm, src_hbm, dst_hbm, att_hbm, zn_hbm,
             numer_out,
             att_v, src_v, dst_v, dd_v, xlr, xrr, tbuf, accb, dbuf,
             numer_sh, gsem, gsem2):
        c = lax.axis_index("c")
        s = lax.axis_index("s")
        w = c * _NS + s

        for j in range(nzc):
            pltpu.sync_copy(zn_hbm.at[pl.ds(0, zr)],
                            numer_sh.at[pl.ds(s * rt + j * zr, zr)])
        for j in range(_cdiv(rtd, zr)):
            rows = min(zr, rtd - j * zr)
            pltpu.sync_copy(zn_hbm.at[pl.ds(0, rows)],
                            numer_sh.at[pl.ds(n_pad + s * rtd + j * zr, rows)])
        pltpu.sync_copy(att_hbm, att_v)
        plsc.subcore_barrier()

        def chunk_body(j, carry):
            base = (w * cpw + j) * _CHUNK
            ci1 = pltpu.async_copy(src_hbm.at[pl.ds(base, _CHUNK)], src_v, gsem)
            ci2 = pltpu.async_copy(dst_hbm.at[pl.ds(base, _CHUNK)], dst_v, gsem2)
            ci1.wait()
            ci2.wait()
            cp1 = pltpu.async_copy(xl_hbm.at[src_v], xlr, gsem)
            cp2 = pltpu.async_copy(xr_hbm.at[dst_v], xrr, gsem2)
            for q in range(_CHUNK // 16):
                dv = dst_v[pl.ds(q * 16, 16)]
                dd_v[pl.ds(q * 16, 16)] = n_pad + lax.shift_right_logical(dv, 3)
            cp1.wait()
            cp2.wait()

            def edge_body(e, carry2):
                acc = jnp.zeros((16,), jnp.float32)
                for k in range(nk):
                    a = xlr[e, pl.ds(k * 16, 16)]
                    r = xrr[e, pl.ds(k * 16, 16)]
                    v = a + r
                    v = jnp.maximum(v, 0.2 * v)
                    acc = acc + v * att_v[pl.ds(k * 16, 16)]
                accb[pl.ds(e * 16, 16)] = acc
                return carry2
            lax.fori_loop(0, _CHUNK, edge_body, 0)

            iota = lax.iota(jnp.int32, 16)

            def group_body(g, carry2):
                rows = (g * 16 + iota) * 16
                tot = jnp.zeros((16,), jnp.float32)
                for col in range(16):
                    tot = tot + plsc.load_gather(accb, [rows + col])
                valid = (base + g * 16 + iota) < e_tot
                t16 = jnp.where(valid, jnp.exp(tot), jnp.float32(0.0))
                dstg = dst_v[pl.ds(g * 16, 16)]
                for el in range(16):
                    t_sc = t16[el]
                    e = g * 16 + el
                    for k in range(nk):
                        xlr[e, pl.ds(k * 16, 16)] = (
                            xlr[e, pl.ds(k * 16, 16)] * t_sc)
                    slot = dstg[el] & 7
                    tb16 = jnp.broadcast_to(t_sc, (16,))
                    for sl in range(8):
                        on = jnp.where(sl == slot, jnp.float32(1.0),
                                       jnp.float32(0.0))
                        tbuf[e, pl.ds(sl * 16, 16)] = tb16 * on
                return carry2
            lax.fori_loop(0, _CHUNK // 16, group_body, 0)
            cs1 = pltpu.async_copy(xlr, numer_sh.at[dst_v], gsem, add=True)
            cs2 = pltpu.async_copy(tbuf, numer_sh.at[dd_v], gsem2, add=True)
            cs1.wait()
            cs2.wait()
            return carry
        lax.fori_loop(0, cpw, chunk_body, 0)
        plsc.subcore_barrier()
        pltpu.sync_copy(numer_sh.at[pl.ds(s * rt, rt)],
                        numer_out.at[pl.ds(c * 2 * n_pad + s * rt, rt)])
        for jj in range(rt // _CHUNK):
            pltpu.sync_copy(
                numer_sh.at[pl.ds(n_pad + s * rtd + jj * (_CHUNK // 8),
                                  _CHUNK // 8)], dbuf)

            def expand_body(n, carry):
                r = lax.shift_right_logical(n, 3)
                sl = n & 7
                dchunk = dbuf[r, pl.ds(sl * 16, 16)]
                b16 = jnp.broadcast_to(dchunk[0], (16,))
                for k in range(nk):
                    tbuf[n, pl.ds(k * 16, 16)] = b16
                return carry
            lax.fori_loop(0, _CHUNK, expand_body, 0)
            pltpu.sync_copy(
                tbuf,
                numer_out.at[pl.ds(
                    c * 2 * n_pad + n_pad + s * rt + jj * _CHUNK, _CHUNK)])

    kfn = pl.kernel(
        body,
        out_type=jax.ShapeDtypeStruct((_NC * 2 * n_pad, d), jnp.float32),
        mesh=_mesh(),
        compiler_params=pltpu.CompilerParams(needs_layout_passes=False),
        scratch_types=[
            pltpu.VMEM((d,), jnp.float32),
            pltpu.VMEM((_CHUNK,), jnp.int32),
            pltpu.VMEM((_CHUNK,), jnp.int32),
            pltpu.VMEM((_CHUNK,), jnp.int32),
            pltpu.VMEM((_CHUNK, d), jnp.float32),
            pltpu.VMEM((_CHUNK, d), jnp.float32),
            pltpu.VMEM((_CHUNK, d), jnp.float32),
            pltpu.VMEM((_CHUNK * 16,), jnp.float32),
            pltpu.VMEM((_CHUNK // 8, d), jnp.float32),
            pltpu.VMEM_SHARED((npd, d), jnp.float32),
            pltpu.SemaphoreType.DMA,
            pltpu.SemaphoreType.DMA,
        ],
    )
    return kfn, e_pad


def _make_pool_kernel():
    pw = _NP_PAD // _NW
    ncn = pw // 16
    nk = _D // 16

    def body(h_hbm, idx_hbm, w_hbm, out_hbm, idx_v, w_v, rows_v, hp_v):
        c = lax.axis_index("c")
        s = lax.axis_index("s")
        wkr = c * _NS + s
        nbase = wkr * pw
        pltpu.sync_copy(w_hbm.at[pl.ds(nbase * _K, pw * _K)], w_v)
        for cn in range(ncn):
            rbase = (nbase + cn * 16) * _K
            pltpu.sync_copy(idx_hbm.at[pl.ds(rbase, 16 * _K)], idx_v)
            pltpu.sync_copy(h_hbm.at[idx_v], rows_v)

            def pair_body(p, carry):
                wv = w_v[pl.ds(cn * 128 + p * 16, 16)]
                for half in range(2):
                    n = 2 * p + half
                    for dk in range(nk):
                        acc = jnp.zeros((16,), jnp.float32)
                        for k in range(_K):
                            acc = acc + (wv[half * _K + k]
                                         * rows_v[n * _K + k, pl.ds(dk * 16, 16)])
                        hp_v[n, pl.ds(dk * 16, 16)] = acc
                return carry
            lax.fori_loop(0, 8, pair_body, 0)
            pltpu.sync_copy(hp_v, out_hbm.at[pl.ds(nbase + cn * 16, 16)])

    return pl.kernel(
        body,
        out_type=jax.ShapeDtypeStruct((_NP_PAD, _D), jnp.float32),
        mesh=_mesh(),
        compiler_params=pltpu.CompilerParams(needs_layout_passes=False),
        scratch_types=[
            pltpu.VMEM((16 * _K,), jnp.int32),
            pltpu.VMEM((pw * _K,), jnp.float32),
            pltpu.VMEM((16 * _K, _D), jnp.float32),
            pltpu.VMEM((16, _D), jnp.float32),
        ],
    )


def _mm2_body(x_ref, wl_ref, wr_ref, xl_ref, xr_ref):
    x = x_ref[...]
    xl_ref[...] = jnp.dot(x, wl_ref[...], preferred_element_type=jnp.float32)
    xr_ref[...] = jnp.dot(x, wr_ref[...], preferred_element_type=jnp.float32)


def _mm2(x, wl, wr):
    n, d = x.shape
    dh = wl.shape[1]
    return pl.pallas_call(
        _mm2_body,
        out_shape=(jax.ShapeDtypeStruct((n, dh), jnp.float32),
                   jax.ShapeDtypeStruct((n, dh), jnp.float32)),
    )(x, wl, wr)


def _unpack(num, n, n_pad):
    outs = []
    for c in range(_NC):
        blk = num[c * 2 * n_pad:(c + 1) * 2 * n_pad]
        outs.append((blk[0:n], blk[n_pad:n_pad + n, 0:1]))
    return outs


def _make_combine_body(n, n_pad):
    def body(num_ref, b_ref, wl_ref, wr_ref, xl_ref, xr_ref):
        (n0, d0), (n1, d1) = _unpack(num_ref[...], n, n_pad)
        h = (n0 + n1) / (d0 + d1) + b_ref[...]
        h = jnp.maximum(h, 0.0)
        xl_ref[...] = jnp.dot(h, wl_ref[...], preferred_element_type=jnp.float32)
        xr_ref[...] = jnp.dot(h, wr_ref[...], preferred_element_type=jnp.float32)
    return body


def _combine(num, b, wl, wr, n, n_pad):
    dh = wl.shape[1]
    return pl.pallas_call(
        _make_combine_body(n, n_pad),
        out_shape=(jax.ShapeDtypeStruct((n, dh), jnp.float32),
                   jax.ShapeDtypeStruct((n, dh), jnp.float32)),
    )(num, b, wl, wr)


def _make_h4_body(n, n_pad):
    def body(num_ref, b_ref, h_ref):
        (n0, d0), (n1, d1) = _unpack(num_ref[...], n, n_pad)
        h = (n0 + n1) / (d0 + d1) + b_ref[...]
        h_ref[...] = jnp.where(h > 0, h, jnp.exp(h) - 1.0)
    return body


def _h4(num, b, n, n_pad, d):
    return pl.pallas_call(
        _make_h4_body(n, n_pad),
        out_shape=jax.ShapeDtypeStruct((n, d), jnp.float32),
    )(num, b)


def _final_body(num_ref, b_ref, batch_ref, gw_ref, gb_ref, cw_ref,
                cb_ref, out_ref):
    (n0, d0), (n1, d1) = _unpack(num_ref[...], _NP, _NP_PAD)
    h = (n0[:, 0:_DP] + n1[:, 0:_DP]) / (d0 + d1) + b_ref[...]
    h = jnp.where(h > 0, h, jnp.exp(h) - 1.0)
    gate = jnp.sum(h * gw_ref[...], axis=1) + gb_ref[0, 0]
    bvec = batch_ref[0, :]
    mask = bvec[None, :] == lax.broadcasted_iota(jnp.int32, (_G, _NP), 0)
    m = jnp.max(jnp.where(mask, gate[None, :], jnp.float32(-1e30)), axis=1)
    m = jnp.where(m < -1e29, 0.0, m)
    e = jnp.where(mask, jnp.exp(gate[None, :] - m[:, None]), 0.0)
    ssum = jnp.sum(e, axis=1)
    wn = e / (ssum[:, None] + 1e-16)
    pooled = jnp.dot(wn, h, preferred_element_type=jnp.float32)
    out_ref[...] = (jnp.dot(pooled, cw_ref[...], preferred_element_type=jnp.float32)
                    + cb_ref[...])


def _final(num, b, batch2, gw, gb, cw, cb):
    return pl.pallas_call(
        _final_body,
        out_shape=jax.ShapeDtypeStruct((_G, cw.shape[1]), jnp.float32),
    )(num, b, batch2, gw, gb, cw, cb)


def kernel(x, edge_index, batch, index_matrix_0, weight_matrix_0,
           pooled_edge_index_0, batch_1,
           gat0_Wl, gat0_Wr, gat0_att, gat0_b,
           gat1_Wl, gat1_Wr, gat1_att, gat1_b,
           gat2_Wl, gat2_Wr, gat2_att, gat2_b,
           gat3_Wl, gat3_Wr, gat3_att, gat3_b,
           gcn_Wl, gcn_Wr, gcn_att, gcn_b,
           gate_W, gate_b, cls_W, cls_b):
    i32 = jnp.int32
    edge_k, e_pad = _make_edge_kernel(_N_PAD, _D, _E_TOT)
    edge_kp, ep_pad = _make_edge_kernel(_NP_PAD, _D, _EP_TOT)
    pool_k = _make_pool_kernel()

    zn = jnp.zeros((_ZRN, _D), jnp.float32)
    si = jnp.arange(_N, dtype=i32)
    zpad = jnp.zeros((e_pad - _E_TOT,), i32)
    src = jnp.concatenate([edge_index[0].astype(i32), si, zpad])
    dst = jnp.concatenate([edge_index[1].astype(i32), si, zpad])

    gat_params = [(gat0_Wl, gat0_Wr, gat0_att, gat0_b),
                  (gat1_Wl, gat1_Wr, gat1_att, gat1_b),
                  (gat2_Wl, gat2_Wr, gat2_att, gat2_b),
                  (gat3_Wl, gat3_Wr, gat3_att, gat3_b)]

    xl, xr = _mm2(x, gat0_Wl, gat0_Wr)
    num = edge_k(xl, xr, src, dst, gat0_att, zn)
    for l in range(1, 4):
        wl, wr = gat_params[l][0], gat_params[l][1]
        b_prev = gat_params[l - 1][3].reshape(1, _D)
        xl, xr = _combine(num, b_prev, wl, wr, _N, _N_PAD)
        num = edge_k(xl, xr, src, dst, gat_params[l][2], zn)
    h = _h4(num, gat3_b.reshape(1, _D), _N, _N_PAD, _D)

    idx_flat = jnp.pad(index_matrix_0.astype(i32),
                       ((0, _NP_PAD - _NP), (0, 0))).reshape(-1)
    w_flat = jnp.pad(weight_matrix_0,
                     ((0, _NP_PAD - _NP), (0, 0))).reshape(-1)
    hp = pool_k(h, idx_flat, w_flat)[:_NP]

    gcn_wl_pad = jnp.pad(gcn_Wl, ((0, 0), (0, _D - _DP)))
    gcn_wr_pad = jnp.pad(gcn_Wr, ((0, 0), (0, _D - _DP)))
    xlp, xrp = _mm2(hp, gcn_wl_pad, gcn_wr_pad)
    sip = jnp.arange(_NP, dtype=i32)
    zpadp = jnp.zeros((ep_pad - _EP_TOT,), i32)
    srcp = jnp.concatenate([pooled_edge_index_0[0].astype(i32), sip, zpadp])
    dstp = jnp.concatenate([pooled_edge_index_0[1].astype(i32), sip, zpadp])
    nump = edge_kp(xlp, xrp, srcp, dstp,
                   jnp.pad(gcn_att, (0, _D - _DP)), zn)

    return _final(nump, gcn_b.reshape(1, _DP),
                  batch_1.astype(i32).reshape(1, _NP),
                  gate_W.reshape(1, _DP), gate_b.reshape(1, 1),
                  cls_W, cls_b.reshape(1, cls_W.shape[1]))
